# baseline (speedup 1.0000x reference)
import math

import jax
import jax.numpy as jnp
from jax.experimental import pallas as pl
from jax.experimental.pallas import tpu as pltpu

_K = 16
_EPS = 1e-8
_T = 512

_SORT8_NET = [(0, 1), (2, 3), (4, 5), (6, 7), (0, 2), (1, 3), (4, 6), (5, 7),
              (1, 2), (5, 6), (0, 4), (1, 5), (2, 6), (3, 7), (2, 4), (3, 5),
              (1, 2), (3, 4), (5, 6)]


def _norm_body(km_ref, kn_ref):
    km = km_ref[0]
    kn_ref[0] = km * jax.lax.rsqrt(
        jnp.maximum(jnp.sum(km * km, axis=-1, keepdims=True), _EPS * _EPS))


_RB = 16


def _body(q_ref, o_ref, kn_ref, vm_ref, g_ref, out_ref, sims_ref, thr_ref):
    q = q_ref[0, 0]
    kn = kn_ref[0]
    vm = vm_ref[0]
    hd = q.shape[-1]

    qn = q * jax.lax.rsqrt(
        jnp.maximum(jnp.sum(q * q, axis=-1, keepdims=True), _EPS * _EPS))

    sims_ref[...] = jax.lax.dot_general(
        qn, kn, (((1,), (1,)), ((), ())),
        preferred_element_type=jnp.float32)

    nc = sims_ref.shape[-1] // 128
    nb = sims_ref.shape[0] // _RB

    def blk(b, carry):
        r0 = pl.ds(b * _RB, _RB)
        cols = [sims_ref[r0, i * 128:(i + 1) * 128] for i in range(nc)]
        for i, j in _SORT8_NET:
            x, y = cols[i], cols[j]
            cols[i] = jnp.maximum(x, y)
            cols[j] = jnp.minimum(x, y)
        neg = jnp.full_like(cols[0], -jnp.inf)
        m = jnp.max(cols[0], axis=-1, keepdims=True)
        for t in range(1, _K):
            msk = cols[0] == m
            for j in range(min(nc, _K - t)):
                nxt = cols[j + 1] if j + 1 < nc else neg
                cols[j] = jnp.where(msk, nxt, cols[j])
            m = jnp.max(cols[0], axis=-1, keepdims=True)
        thr_ref[r0, :] = m
        return carry

    jax.lax.fori_loop(0, nb, blk, 0, unroll=False)

    sims = sims_ref[...]
    w = jnp.where(sims >= thr_ref[...], sims, 0.0)
    wm = jax.lax.dot_general(
        w, vm, (((1,), (0,)), ((), ())),
        preferred_element_type=jnp.float32)

    g = jax.nn.sigmoid(g_ref[0, 0, 0])
    out_ref[0, 0] = (g / math.sqrt(hd)) * wm + (1.0 - g) * o_ref[0, 0]


def kernel(inputs, query, key, value, outputs, gate, key_memories, value_memories):
    del inputs, key, value
    B, H, S, HD = query.shape
    M = key_memories.shape[1]
    nt = S // _T
    grid = (H, B * nt)

    kn = pl.pallas_call(
        _norm_body,
        grid=(H,),
        in_specs=[pl.BlockSpec((1, M, HD), lambda h: (h, 0, 0))],
        out_specs=pl.BlockSpec((1, M, HD), lambda h: (h, 0, 0)),
        out_shape=jax.ShapeDtypeStruct((H, M, HD), jnp.float32),
    )(key_memories)

    gate2 = gate.reshape(H, 1, 1)

    out = pl.pallas_call(
        _body,
        grid=grid,
        in_specs=[
            pl.BlockSpec((1, 1, _T, HD), lambda h, t: (t // nt, h, t % nt, 0)),
            pl.BlockSpec((1, 1, _T, HD), lambda h, t: (t // nt, h, t % nt, 0)),
            pl.BlockSpec((1, M, HD), lambda h, t: (h, 0, 0)),
            pl.BlockSpec((1, M, HD), lambda h, t: (h, 0, 0)),
            pl.BlockSpec((1, 1, 1), lambda h, t: (h, 0, 0)),
        ],
        out_specs=pl.BlockSpec((1, 1, _T, HD), lambda h, t: (t // nt, h, t % nt, 0)),
        out_shape=jax.ShapeDtypeStruct((B, H, S, HD), jnp.float32),
        scratch_shapes=[
            pltpu.VMEM((_T, M), jnp.float32),
            pltpu.VMEM((_T, 1), jnp.float32),
        ],
    )(query, outputs, kn, value_memories, gate2)
    return out


# speedup vs baseline: 10.0330x; 10.0330x over previous
import math

import jax
import jax.numpy as jnp
from jax.experimental import pallas as pl
from jax.experimental.pallas import tpu as pltpu

_K = 16
_EPS = 1e-8
_T = 512

_SORT8_NET = [(0, 1), (2, 3), (4, 5), (6, 7), (0, 2), (1, 3), (4, 6), (5, 7),
              (1, 2), (5, 6), (0, 4), (1, 5), (2, 6), (3, 7), (2, 4), (3, 5),
              (1, 2), (3, 4), (5, 6)]


def _norm_body(km_ref, kn_ref):
    km = km_ref[0]
    kn_ref[0] = km * jax.lax.rsqrt(
        jnp.maximum(jnp.sum(km * km, axis=-1, keepdims=True), _EPS * _EPS))


_RB = 16


def _body(q_ref, o_ref, kn_ref, vm_ref, g_ref, out_ref):
    q = q_ref[0, 0]
    kn = kn_ref[0]
    vm = vm_ref[0]
    hd = q.shape[-1]

    qn = q * jax.lax.rsqrt(
        jnp.maximum(jnp.sum(q * q, axis=-1, keepdims=True), _EPS * _EPS))

    sims = jax.lax.dot_general(
        qn, kn, (((1,), (1,)), ((), ())),
        preferred_element_type=jnp.float32)

    nc = sims.shape[-1] // 128
    nb = sims.shape[0] // _RB

    w_parts = []
    for b in range(nb):
        s = sims[b * _RB:(b + 1) * _RB, :]
        cols = [s[:, i * 128:(i + 1) * 128] for i in range(nc)]
        for i, j in _SORT8_NET:
            x, y = cols[i], cols[j]
            cols[i] = jnp.maximum(x, y)
            cols[j] = jnp.minimum(x, y)
        neg = jnp.full_like(cols[0], -jnp.inf)
        m = jnp.max(cols[0], axis=-1, keepdims=True)
        for t in range(1, _K):
            msk = cols[0] == m
            for j in range(min(nc, _K - t)):
                nxt = cols[j + 1] if j + 1 < nc else neg
                cols[j] = jnp.where(msk, nxt, cols[j])
            m = jnp.max(cols[0], axis=-1, keepdims=True)
        w_parts.append(jnp.where(s >= m, s, 0.0))

    w = jnp.concatenate(w_parts, axis=0)
    wm = jax.lax.dot_general(
        w, vm, (((1,), (0,)), ((), ())),
        preferred_element_type=jnp.float32)

    g = jax.nn.sigmoid(g_ref[0, 0, 0])
    out_ref[0, 0] = (g / math.sqrt(hd)) * wm + (1.0 - g) * o_ref[0, 0]


def kernel(inputs, query, key, value, outputs, gate, key_memories, value_memories):
    del inputs, key, value
    B, H, S, HD = query.shape
    M = key_memories.shape[1]
    nt = S // _T
    grid = (H, B * nt)

    kn = pl.pallas_call(
        _norm_body,
        grid=(H,),
        in_specs=[pl.BlockSpec((1, M, HD), lambda h: (h, 0, 0))],
        out_specs=pl.BlockSpec((1, M, HD), lambda h: (h, 0, 0)),
        out_shape=jax.ShapeDtypeStruct((H, M, HD), jnp.float32),
    )(key_memories)

    gate2 = gate.reshape(H, 1, 1)

    out = pl.pallas_call(
        _body,
        grid=grid,
        in_specs=[
            pl.BlockSpec((1, 1, _T, HD), lambda h, t: (t // nt, h, t % nt, 0)),
            pl.BlockSpec((1, 1, _T, HD), lambda h, t: (t // nt, h, t % nt, 0)),
            pl.BlockSpec((1, M, HD), lambda h, t: (h, 0, 0)),
            pl.BlockSpec((1, M, HD), lambda h, t: (h, 0, 0)),
            pl.BlockSpec((1, 1, 1), lambda h, t: (h, 0, 0)),
        ],
        out_specs=pl.BlockSpec((1, 1, _T, HD), lambda h, t: (t // nt, h, t % nt, 0)),
        out_shape=jax.ShapeDtypeStruct((B, H, S, HD), jnp.float32),
    )(query, outputs, kn, value_memories, gate2)
    return out


# speedup vs baseline: 10.2566x; 1.0223x over previous
import math

import jax
import jax.numpy as jnp
from jax.experimental import pallas as pl
from jax.experimental.pallas import tpu as pltpu

_K = 16
_EPS = 1e-8
_T = 512

_SORT8_NET = [(0, 1), (2, 3), (4, 5), (6, 7), (0, 2), (1, 3), (4, 6), (5, 7),
              (1, 2), (5, 6), (0, 4), (1, 5), (2, 6), (3, 7), (2, 4), (3, 5),
              (1, 2), (3, 4), (5, 6)]


def _norm_body(km_ref, kn_ref):
    km = km_ref[0]
    kn_ref[0] = km * jax.lax.rsqrt(
        jnp.maximum(jnp.sum(km * km, axis=-1, keepdims=True), _EPS * _EPS))


_RB = 8


def _body(q_ref, o_ref, kn_ref, vm_ref, g_ref, out_ref):
    q = q_ref[0, 0]
    kn = kn_ref[0]
    vm = vm_ref[0]
    hd = q.shape[-1]

    qn = q * jax.lax.rsqrt(
        jnp.maximum(jnp.sum(q * q, axis=-1, keepdims=True), _EPS * _EPS))

    sims = jax.lax.dot_general(
        qn, kn, (((1,), (1,)), ((), ())),
        preferred_element_type=jnp.float32)

    nc = sims.shape[-1] // 128
    nb = sims.shape[0] // _RB

    w_parts = []
    for b in range(nb):
        s = sims[b * _RB:(b + 1) * _RB, :]
        cols = [s[:, i * 128:(i + 1) * 128] for i in range(nc)]
        for i, j in _SORT8_NET:
            x, y = cols[i], cols[j]
            cols[i] = jnp.maximum(x, y)
            cols[j] = jnp.minimum(x, y)
        neg = jnp.full_like(cols[0], -jnp.inf)
        c = cols + [neg]
        h = cols[0]
        m = jnp.max(h, axis=-1, keepdims=True)
        for t in range(1, _K):
            b1 = c[4] < m
            b2 = jnp.where(b1, c[2], c[6]) < m
            b3 = jnp.where(b1, jnp.where(b2, c[1], c[3]),
                           jnp.where(b2, c[5], c[7])) < m
            nv = jnp.where(
                b1,
                jnp.where(b2, jnp.where(b3, c[1], c[2]),
                          jnp.where(b3, c[3], c[4])),
                jnp.where(b2, jnp.where(b3, c[5], c[6]),
                          jnp.where(b3, c[7], c[8])))
            h = jnp.where(h == m, nv, h)
            m = jnp.max(h, axis=-1, keepdims=True)
        w_parts.append(jnp.where(s >= m, s, 0.0))

    w = jnp.concatenate(w_parts, axis=0)
    wm = jax.lax.dot_general(
        w, vm, (((1,), (0,)), ((), ())),
        preferred_element_type=jnp.float32)

    g = jax.nn.sigmoid(g_ref[0, 0, 0])
    out_ref[0, 0] = (g / math.sqrt(hd)) * wm + (1.0 - g) * o_ref[0, 0]


def kernel(inputs, query, key, value, outputs, gate, key_memories, value_memories):
    del inputs, key, value
    B, H, S, HD = query.shape
    M = key_memories.shape[1]
    nt = S // _T
    grid = (H, B * nt)

    kn = pl.pallas_call(
        _norm_body,
        grid=(H,),
        in_specs=[pl.BlockSpec((1, M, HD), lambda h: (h, 0, 0))],
        out_specs=pl.BlockSpec((1, M, HD), lambda h: (h, 0, 0)),
        out_shape=jax.ShapeDtypeStruct((H, M, HD), jnp.float32),
    )(key_memories)

    gate2 = gate.reshape(H, 1, 1)

    out = pl.pallas_call(
        _body,
        grid=grid,
        in_specs=[
            pl.BlockSpec((1, 1, _T, HD), lambda h, t: (t // nt, h, t % nt, 0)),
            pl.BlockSpec((1, 1, _T, HD), lambda h, t: (t // nt, h, t % nt, 0)),
            pl.BlockSpec((1, M, HD), lambda h, t: (h, 0, 0)),
            pl.BlockSpec((1, M, HD), lambda h, t: (h, 0, 0)),
            pl.BlockSpec((1, 1, 1), lambda h, t: (h, 0, 0)),
        ],
        out_specs=pl.BlockSpec((1, 1, _T, HD), lambda h, t: (t // nt, h, t % nt, 0)),
        out_shape=jax.ShapeDtypeStruct((B, H, S, HD), jnp.float32),
    )(query, outputs, kn, value_memories, gate2)
    return out


# speedup vs baseline: 11.0167x; 1.0741x over previous
import math

import jax
import jax.numpy as jnp
from jax.experimental import pallas as pl
from jax.experimental.pallas import tpu as pltpu

_K = 16
_EPS = 1e-8
_T = 1024

_SORT8_NET = [(0, 1), (2, 3), (4, 5), (6, 7), (0, 2), (1, 3), (4, 6), (5, 7),
              (1, 2), (5, 6), (0, 4), (1, 5), (2, 6), (3, 7), (2, 4), (3, 5),
              (1, 2), (3, 4), (5, 6)]


def _norm_body(km_ref, kn_ref):
    km = km_ref[0]
    kn_ref[0] = km * jax.lax.rsqrt(
        jnp.maximum(jnp.sum(km * km, axis=-1, keepdims=True), _EPS * _EPS))


_RB = 8


def _body(q_ref, o_ref, kn_ref, vm_ref, g_ref, out_ref):
    q = q_ref[0, 0]
    kn = kn_ref[0]
    vm = vm_ref[0]
    hd = q.shape[-1]

    qn = q * jax.lax.rsqrt(
        jnp.maximum(jnp.sum(q * q, axis=-1, keepdims=True), _EPS * _EPS))

    sims = jax.lax.dot_general(
        qn, kn, (((1,), (1,)), ((), ())),
        preferred_element_type=jnp.float32)

    nc = sims.shape[-1] // 128
    nb = sims.shape[0] // _RB

    w_parts = []
    for b in range(nb):
        s = sims[b * _RB:(b + 1) * _RB, :]
        cols = [s[:, i * 128:(i + 1) * 128] for i in range(nc)]
        for i, j in _SORT8_NET:
            x, y = cols[i], cols[j]
            cols[i] = jnp.maximum(x, y)
            cols[j] = jnp.minimum(x, y)
        neg = jnp.full_like(cols[0], -jnp.inf)
        c = cols + [neg]
        h = cols[0]
        m = jnp.max(h, axis=-1, keepdims=True)
        for t in range(1, _K):
            b1 = c[4] < m
            b2 = jnp.where(b1, c[2], c[6]) < m
            b3 = jnp.where(b1, jnp.where(b2, c[1], c[3]),
                           jnp.where(b2, c[5], c[7])) < m
            nv = jnp.where(
                b1,
                jnp.where(b2, jnp.where(b3, c[1], c[2]),
                          jnp.where(b3, c[3], c[4])),
                jnp.where(b2, jnp.where(b3, c[5], c[6]),
                          jnp.where(b3, c[7], c[8])))
            h = jnp.where(h == m, nv, h)
            m = jnp.max(h, axis=-1, keepdims=True)
        w_parts.append(jnp.where(s >= m, s, 0.0))

    w = jnp.concatenate(w_parts, axis=0)
    wm = jax.lax.dot_general(
        w, vm, (((1,), (0,)), ((), ())),
        preferred_element_type=jnp.float32)

    g = jax.nn.sigmoid(g_ref[0, 0, 0])
    out_ref[0, 0] = (g / math.sqrt(hd)) * wm + (1.0 - g) * o_ref[0, 0]


def kernel(inputs, query, key, value, outputs, gate, key_memories, value_memories):
    del inputs, key, value
    B, H, S, HD = query.shape
    M = key_memories.shape[1]
    nt = S // _T
    grid = (H, B * nt)

    kn = pl.pallas_call(
        _norm_body,
        grid=(H,),
        in_specs=[pl.BlockSpec((1, M, HD), lambda h: (h, 0, 0))],
        out_specs=pl.BlockSpec((1, M, HD), lambda h: (h, 0, 0)),
        out_shape=jax.ShapeDtypeStruct((H, M, HD), jnp.float32),
    )(key_memories)

    gate2 = gate.reshape(H, 1, 1)

    out = pl.pallas_call(
        _body,
        grid=grid,
        in_specs=[
            pl.BlockSpec((1, 1, _T, HD), lambda h, t: (t // nt, h, t % nt, 0)),
            pl.BlockSpec((1, 1, _T, HD), lambda h, t: (t // nt, h, t % nt, 0)),
            pl.BlockSpec((1, M, HD), lambda h, t: (h, 0, 0)),
            pl.BlockSpec((1, M, HD), lambda h, t: (h, 0, 0)),
            pl.BlockSpec((1, 1, 1), lambda h, t: (h, 0, 0)),
        ],
        out_specs=pl.BlockSpec((1, 1, _T, HD), lambda h, t: (t // nt, h, t % nt, 0)),
        out_shape=jax.ShapeDtypeStruct((B, H, S, HD), jnp.float32),
    )(query, outputs, kn, value_memories, gate2)
    return out


# speedup vs baseline: 11.2371x; 1.0200x over previous
import math

import jax
import jax.numpy as jnp
from jax.experimental import pallas as pl
from jax.experimental.pallas import tpu as pltpu

_K = 16
_EPS = 1e-8
_T = 2048

_SORT8_NET = [(0, 1), (2, 3), (4, 5), (6, 7), (0, 2), (1, 3), (4, 6), (5, 7),
              (1, 2), (5, 6), (0, 4), (1, 5), (2, 6), (3, 7), (2, 4), (3, 5),
              (1, 2), (3, 4), (5, 6)]


def _norm_body(km_ref, kn_ref):
    km = km_ref[0]
    kn_ref[0] = km * jax.lax.rsqrt(
        jnp.maximum(jnp.sum(km * km, axis=-1, keepdims=True), _EPS * _EPS))


_RB = 8


def _body(q_ref, o_ref, kn_ref, vm_ref, g_ref, out_ref):
    q = q_ref[0, 0]
    kn = kn_ref[0]
    vm = vm_ref[0]
    hd = q.shape[-1]

    qn = q * jax.lax.rsqrt(
        jnp.maximum(jnp.sum(q * q, axis=-1, keepdims=True), _EPS * _EPS))

    sims = jax.lax.dot_general(
        qn, kn, (((1,), (1,)), ((), ())),
        preferred_element_type=jnp.float32)

    nc = sims.shape[-1] // 128
    nb = sims.shape[0] // _RB

    w_parts = []
    for b in range(nb):
        s = sims[b * _RB:(b + 1) * _RB, :]
        cols = [s[:, i * 128:(i + 1) * 128] for i in range(nc)]
        for i, j in _SORT8_NET:
            x, y = cols[i], cols[j]
            cols[i] = jnp.maximum(x, y)
            cols[j] = jnp.minimum(x, y)
        neg = jnp.full_like(cols[0], -jnp.inf)
        c = cols + [neg]
        h = cols[0]
        m = jnp.max(h, axis=-1, keepdims=True)
        for t in range(1, _K):
            b1 = c[4] < m
            b2 = jnp.where(b1, c[2], c[6]) < m
            b3 = jnp.where(b1, jnp.where(b2, c[1], c[3]),
                           jnp.where(b2, c[5], c[7])) < m
            nv = jnp.where(
                b1,
                jnp.where(b2, jnp.where(b3, c[1], c[2]),
                          jnp.where(b3, c[3], c[4])),
                jnp.where(b2, jnp.where(b3, c[5], c[6]),
                          jnp.where(b3, c[7], c[8])))
            h = jnp.where(h == m, nv, h)
            m = jnp.max(h, axis=-1, keepdims=True)
        w_parts.append(jnp.where(s >= m, s, 0.0))

    w = jnp.concatenate(w_parts, axis=0)
    wm = jax.lax.dot_general(
        w, vm, (((1,), (0,)), ((), ())),
        preferred_element_type=jnp.float32)

    g = jax.nn.sigmoid(g_ref[0, 0, 0])
    out_ref[0, 0] = (g / math.sqrt(hd)) * wm + (1.0 - g) * o_ref[0, 0]


def kernel(inputs, query, key, value, outputs, gate, key_memories, value_memories):
    del inputs, key, value
    B, H, S, HD = query.shape
    M = key_memories.shape[1]
    nt = S // _T
    grid = (H, B * nt)

    kn = pl.pallas_call(
        _norm_body,
        grid=(H,),
        in_specs=[pl.BlockSpec((1, M, HD), lambda h: (h, 0, 0))],
        out_specs=pl.BlockSpec((1, M, HD), lambda h: (h, 0, 0)),
        out_shape=jax.ShapeDtypeStruct((H, M, HD), jnp.float32),
    )(key_memories)

    gate2 = gate.reshape(H, 1, 1)

    out = pl.pallas_call(
        _body,
        grid=grid,
        in_specs=[
            pl.BlockSpec((1, 1, _T, HD), lambda h, t: (t // nt, h, t % nt, 0)),
            pl.BlockSpec((1, 1, _T, HD), lambda h, t: (t // nt, h, t % nt, 0)),
            pl.BlockSpec((1, M, HD), lambda h, t: (h, 0, 0)),
            pl.BlockSpec((1, M, HD), lambda h, t: (h, 0, 0)),
            pl.BlockSpec((1, 1, 1), lambda h, t: (h, 0, 0)),
        ],
        out_specs=pl.BlockSpec((1, 1, _T, HD), lambda h, t: (t // nt, h, t % nt, 0)),
        out_shape=jax.ShapeDtypeStruct((B, H, S, HD), jnp.float32),
    )(query, outputs, kn, value_memories, gate2)
    return out


# speedup vs baseline: 11.3959x; 1.0141x over previous
import math

import jax
import jax.numpy as jnp
from jax.experimental import pallas as pl
from jax.experimental.pallas import tpu as pltpu

_K = 16
_EPS = 1e-8
_T = 2048

_SORT8_NET = [(0, 1), (2, 3), (4, 5), (6, 7), (0, 2), (1, 3), (4, 6), (5, 7),
              (1, 2), (5, 6), (0, 4), (1, 5), (2, 6), (3, 7), (2, 4), (3, 5),
              (1, 2), (3, 4), (5, 6)]


_RB = 8


def _body(q_ref, o_ref, km_ref, vm_ref, g_ref, out_ref, kn_ref):
    q = q_ref[0, 0]
    vm = vm_ref[0]
    hd = q.shape[-1]

    @pl.when(pl.program_id(1) == 0)
    def _():
        km = km_ref[0]
        kn_ref[...] = km * jax.lax.rsqrt(
            jnp.maximum(jnp.sum(km * km, axis=-1, keepdims=True), _EPS * _EPS))

    kn = kn_ref[...]

    qn = q * jax.lax.rsqrt(
        jnp.maximum(jnp.sum(q * q, axis=-1, keepdims=True), _EPS * _EPS))

    sims = jax.lax.dot_general(
        qn, kn, (((1,), (1,)), ((), ())),
        preferred_element_type=jnp.float32)

    nc = sims.shape[-1] // 128
    nb = sims.shape[0] // _RB

    w_parts = []
    for b in range(nb):
        s = sims[b * _RB:(b + 1) * _RB, :]
        cols = [s[:, i * 128:(i + 1) * 128] for i in range(nc)]
        for i, j in _SORT8_NET:
            x, y = cols[i], cols[j]
            cols[i] = jnp.maximum(x, y)
            cols[j] = jnp.minimum(x, y)
        neg = jnp.full_like(cols[0], -jnp.inf)
        c = cols + [neg]
        h = cols[0]
        m = jnp.max(h, axis=-1, keepdims=True)
        for t in range(1, _K):
            b1 = c[4] < m
            b2 = jnp.where(b1, c[2], c[6]) < m
            b3 = jnp.where(b1, jnp.where(b2, c[1], c[3]),
                           jnp.where(b2, c[5], c[7])) < m
            nv = jnp.where(
                b1,
                jnp.where(b2, jnp.where(b3, c[1], c[2]),
                          jnp.where(b3, c[3], c[4])),
                jnp.where(b2, jnp.where(b3, c[5], c[6]),
                          jnp.where(b3, c[7], c[8])))
            h = jnp.where(h == m, nv, h)
            m = jnp.max(h, axis=-1, keepdims=True)
        w_parts.append(jnp.where(s >= m, s, 0.0))

    w = jnp.concatenate(w_parts, axis=0)
    wm = jax.lax.dot_general(
        w, vm, (((1,), (0,)), ((), ())),
        preferred_element_type=jnp.float32)

    g = jax.nn.sigmoid(g_ref[0, 0, 0])
    out_ref[0, 0] = (g / math.sqrt(hd)) * wm + (1.0 - g) * o_ref[0, 0]


def kernel(inputs, query, key, value, outputs, gate, key_memories, value_memories):
    del inputs, key, value
    B, H, S, HD = query.shape
    M = key_memories.shape[1]
    nt = S // _T
    grid = (H, B * nt)

    gate2 = gate.reshape(H, 1, 1)

    out = pl.pallas_call(
        _body,
        grid=grid,
        in_specs=[
            pl.BlockSpec((1, 1, _T, HD), lambda h, t: (t // nt, h, t % nt, 0)),
            pl.BlockSpec((1, 1, _T, HD), lambda h, t: (t // nt, h, t % nt, 0)),
            pl.BlockSpec((1, M, HD), lambda h, t: (h, 0, 0)),
            pl.BlockSpec((1, M, HD), lambda h, t: (h, 0, 0)),
            pl.BlockSpec((1, 1, 1), lambda h, t: (h, 0, 0)),
        ],
        out_specs=pl.BlockSpec((1, 1, _T, HD), lambda h, t: (t // nt, h, t % nt, 0)),
        out_shape=jax.ShapeDtypeStruct((B, H, S, HD), jnp.float32),
        scratch_shapes=[pltpu.VMEM((M, HD), jnp.float32)],
    )(query, outputs, key_memories, value_memories, gate2)
    return out
